# Initial kernel scaffold; baseline (speedup 1.0000x reference)
import jax
import jax.numpy as jnp
from jax.experimental import pallas as pl


def kernel(x, edge_index, edge_attr, batch, externals, W_node, b_node, We1, be1, We2, be2, Wc1, bc1, Wc2, bc2, gamma, beta, Wx1, bx1, Wx2, bx2, Wf1, bf1, Wf2, bf2):
    raise NotImplementedError("write your pallas kernel here")


# speedup vs baseline: 3.0669x; 3.0669x over previous
import functools

import jax
import jax.numpy as jnp
from jax import lax
from jax.experimental import pallas as pl
from jax.experimental.pallas import tpu as pltpu
from jax.experimental.pallas import tpu_sc as plsc

N = 10000
E = 320000
G = 256
H = 128
F32 = jnp.float32

_NT = 5
_NROW = N // _NT
_ET = 125
_EROW = E // _ET

_CHUNK = 128
_NCHUNK = E // _CHUNK
_CPC = _NCHUNK // 2
_RPT = 624


def _mm_bias_kernel(x_ref, w_ref, b_ref, o_ref):
    o_ref[...] = jnp.dot(x_ref[...], w_ref[...],
                         preferred_element_type=F32) + b_ref[...]


def _node_embed(x, w, b):
    return pl.pallas_call(
        _mm_bias_kernel,
        grid=(_NT,),
        in_specs=[
            pl.BlockSpec((_NROW, H), lambda i: (i, 0)),
            pl.BlockSpec((H, H), lambda i: (0, 0)),
            pl.BlockSpec((1, H), lambda i: (0, 0)),
        ],
        out_specs=pl.BlockSpec((_NROW, H), lambda i: (i, 0)),
        out_shape=jax.ShapeDtypeStruct((N, H), F32),
    )(x, w, b.reshape(1, H))


def _edge_mlp_kernel(a_ref, w1_ref, b1_ref, w2_ref, b2_ref, o_ref):
    t = jnp.maximum(jnp.dot(a_ref[...], w1_ref[...],
                            preferred_element_type=F32) + b1_ref[...], 0.0)
    o_ref[...] = jnp.dot(t, w2_ref[...],
                         preferred_element_type=F32) + b2_ref[...]


def _edge_mlp(a, w1, b1, w2, b2):
    d = a.shape[1]
    return pl.pallas_call(
        _edge_mlp_kernel,
        grid=(_ET,),
        in_specs=[
            pl.BlockSpec((_EROW, d), lambda i: (i, 0)),
            pl.BlockSpec((d, H), lambda i: (0, 0)),
            pl.BlockSpec((1, H), lambda i: (0, 0)),
            pl.BlockSpec((H, H), lambda i: (0, 0)),
            pl.BlockSpec((1, H), lambda i: (0, 0)),
        ],
        out_specs=pl.BlockSpec((_EROW, H), lambda i: (i, 0)),
        out_shape=jax.ShapeDtypeStruct((E, H), F32),
    )(a, w1, b1.reshape(1, H), w2, b2.reshape(1, H))


def _node_layer_kernel(h_ref, p0_ref, p1_ref, w1_ref, b1_ref, w2_ref, b2_ref,
                       t_ref, stats_ref, ssum, ssq):
    i = pl.program_id(0)

    @pl.when(i == 0)
    def _():
        ssum[...] = jnp.zeros_like(ssum)
        ssq[...] = jnp.zeros_like(ssq)

    z = h_ref[...] + p0_ref[...] + p1_ref[...]
    t = jnp.maximum(jnp.dot(z, w1_ref[...],
                            preferred_element_type=F32) + b1_ref[...], 0.0)
    t = jnp.dot(t, w2_ref[...], preferred_element_type=F32) + b2_ref[...]
    t_ref[...] = t
    ssum[...] += jnp.sum(t, axis=0, keepdims=True)
    ssq[...] += jnp.sum(t * t, axis=0, keepdims=True)

    @pl.when(i == _NT - 1)
    def _():
        stats_ref[0:1, :] = ssum[...]
        stats_ref[1:2, :] = ssq[...]


def _node_layer(h, p0, p1, w1, b1, w2, b2):
    return pl.pallas_call(
        _node_layer_kernel,
        grid=(_NT,),
        in_specs=[
            pl.BlockSpec((_NROW, H), lambda i: (i, 0)),
            pl.BlockSpec((_NROW, H), lambda i: (i, 0)),
            pl.BlockSpec((_NROW, H), lambda i: (i, 0)),
            pl.BlockSpec((H, H), lambda i: (0, 0)),
            pl.BlockSpec((1, H), lambda i: (0, 0)),
            pl.BlockSpec((H, H), lambda i: (0, 0)),
            pl.BlockSpec((1, H), lambda i: (0, 0)),
        ],
        out_specs=[
            pl.BlockSpec((_NROW, H), lambda i: (i, 0)),
            pl.BlockSpec((2, H), lambda i: (0, 0)),
        ],
        out_shape=[
            jax.ShapeDtypeStruct((N, H), F32),
            jax.ShapeDtypeStruct((2, H), F32),
        ],
        scratch_shapes=[
            pltpu.VMEM((1, H), F32),
            pltpu.VMEM((1, H), F32),
        ],
    )(h, p0, p1, w1, b1.reshape(1, H), w2, b2.reshape(1, H))


def _bn_relu_kernel(t_ref, stats_ref, g_ref, b_ref, o_ref):
    mu = stats_ref[0:1, :] * (1.0 / N)
    var = stats_ref[1:2, :] * (1.0 / N) - mu * mu
    inv = lax.rsqrt(var + 1e-5)
    o_ref[...] = jnp.maximum(
        g_ref[...] * (t_ref[...] - mu) * inv + b_ref[...], 0.0)


def _bn_relu(t, stats, g, b):
    return pl.pallas_call(
        _bn_relu_kernel,
        grid=(_NT,),
        in_specs=[
            pl.BlockSpec((_NROW, H), lambda i: (i, 0)),
            pl.BlockSpec((2, H), lambda i: (0, 0)),
            pl.BlockSpec((1, H), lambda i: (0, 0)),
            pl.BlockSpec((1, H), lambda i: (0, 0)),
        ],
        out_specs=pl.BlockSpec((_NROW, H), lambda i: (i, 0)),
        out_shape=jax.ShapeDtypeStruct((N, H), F32),
    )(t, stats, g.reshape(1, H), b.reshape(1, H))


def _final_kernel(h_ref, batch_ref, ext_ref, wx1_ref, bx1_ref, wx2_ref,
                  bx2_ref, wf1_ref, bf1_ref, wf2_ref, bf2_ref,
                  o_ref, sums, cnts):
    i = pl.program_id(0)

    @pl.when(i == 0)
    def _():
        sums[...] = jnp.zeros_like(sums)
        cnts[...] = jnp.zeros_like(cnts)

    b = batch_ref[0]
    bb = jnp.broadcast_to(b, (G, _NROW))
    gi = lax.broadcasted_iota(jnp.int32, (G, _NROW), 0)
    oh = (bb == gi).astype(F32)
    sums[...] += jnp.dot(oh, h_ref[...], preferred_element_type=F32)
    cnts[...] += jnp.dot(oh, jnp.ones((_NROW, H), F32),
                         preferred_element_type=F32)

    @pl.when(i == _NT - 1)
    def _():
        emb = sums[...] / jnp.maximum(cnts[...], 1.0)
        ext = jnp.maximum(jnp.dot(ext_ref[...], wx1_ref[...],
                                  preferred_element_type=F32)
                          + bx1_ref[...], 0.0)
        ext = jnp.dot(ext, wx2_ref[...],
                      preferred_element_type=F32) + bx2_ref[...]
        comb = jnp.concatenate([emb, ext], axis=1)
        r = jnp.maximum(jnp.dot(comb, wf1_ref[...],
                                preferred_element_type=F32)
                        + bf1_ref[...], 0.0)
        o_ref[...] = jnp.dot(r, wf2_ref[...],
                             preferred_element_type=F32) + bf2_ref[...]


def _final(h, batch, ext, wx1, bx1, wx2, bx2, wf1, bf1, wf2, bf2):
    d = ext.shape[1]
    batch3 = batch.reshape(_NT, 1, _NROW)
    return pl.pallas_call(
        _final_kernel,
        grid=(_NT,),
        in_specs=[
            pl.BlockSpec((_NROW, H), lambda i: (i, 0)),
            pl.BlockSpec((1, 1, _NROW), lambda i: (i, 0, 0)),
            pl.BlockSpec((G, d), lambda i: (0, 0)),
            pl.BlockSpec((d, H), lambda i: (0, 0)),
            pl.BlockSpec((1, H), lambda i: (0, 0)),
            pl.BlockSpec((H, H), lambda i: (0, 0)),
            pl.BlockSpec((1, H), lambda i: (0, 0)),
            pl.BlockSpec((2 * H, H), lambda i: (0, 0)),
            pl.BlockSpec((1, H), lambda i: (0, 0)),
            pl.BlockSpec((H, 1), lambda i: (0, 0)),
            pl.BlockSpec((1, 1), lambda i: (0, 0)),
        ],
        out_specs=pl.BlockSpec((G, 1), lambda i: (0, 0)),
        out_shape=jax.ShapeDtypeStruct((G, 1), F32),
        scratch_shapes=[
            pltpu.VMEM((G, H), F32),
            pltpu.VMEM((G, H), F32),
        ],
    )(h, batch3, ext, wx1, bx1.reshape(1, H), wx2, bx2.reshape(1, H),
      wf1, bf1.reshape(1, H), wf2, bf2.reshape(1, 1))


def _edge_stage(h, e, src, dst):
    mesh = plsc.VectorSubcoreMesh(core_axis_name="c", subcore_axis_name="s")

    @functools.partial(
        pl.kernel,
        out_type=jax.ShapeDtypeStruct((2, N, H), F32),
        mesh=mesh,
        scratch_types=[
            pltpu.VMEM((_CHUNK,), jnp.int32),
            pltpu.VMEM((_CHUNK,), jnp.int32),
            pltpu.VMEM((_CHUNK, H), F32),
            pltpu.VMEM((_CHUNK, H), F32),
            pltpu.VMEM_SHARED((N, H), F32),
            pltpu.SemaphoreType.DMA,
        ],
    )
    def k(h_hbm, e_hbm, src_hbm, dst_hbm, out_hbm,
          src_v, dst_v, rows_v, e_v, acc, sem):
        cid = lax.axis_index("c")
        sid = lax.axis_index("s")

        @pl.loop(0, _CHUNK)
        def _(r):
            for j in range(H // 16):
                rows_v[r, pl.ds(j * 16, 16)] = jnp.zeros((16,), F32)
        row0 = sid * _RPT
        for t in range(4):
            pltpu.sync_copy(rows_v, acc.at[pl.ds(row0 + t * _CHUNK, _CHUNK)])
        pltpu.sync_copy(rows_v.at[pl.ds(0, _RPT - 4 * _CHUNK)],
                        acc.at[pl.ds(row0 + 4 * _CHUNK, _RPT - 4 * _CHUNK)])

        @pl.when(sid == 15)
        def _():
            pltpu.sync_copy(rows_v.at[pl.ds(0, N - 16 * _RPT)],
                            acc.at[pl.ds(16 * _RPT, N - 16 * _RPT)])
        plsc.subcore_barrier()

        @pl.loop(sid, _CPC, step=16)
        def _(k0):
            base = (cid * _CPC + k0) * _CHUNK
            pltpu.sync_copy(src_hbm.at[pl.ds(base, _CHUNK)], src_v)
            pltpu.sync_copy(dst_hbm.at[pl.ds(base, _CHUNK)], dst_v)
            pltpu.async_copy(h_hbm.at[src_v], rows_v, sem).wait()
            pltpu.sync_copy(e_hbm.at[pl.ds(base, _CHUNK)], e_v)

            @pl.loop(0, _CHUNK)
            def _(r):
                for j in range(H // 16):
                    sl = pl.ds(j * 16, 16)
                    rows_v[r, sl] = jnp.maximum(rows_v[r, sl] + e_v[r, sl],
                                                0.0)

            pltpu.sync_copy(rows_v, acc.at[dst_v], add=True)

        plsc.subcore_barrier()
        pltpu.sync_copy(acc.at[pl.ds(row0, _RPT)],
                        out_hbm.at[cid].at[pl.ds(row0, _RPT)])

        @pl.when(sid == 15)
        def _():
            pltpu.sync_copy(acc.at[pl.ds(16 * _RPT, N - 16 * _RPT)],
                            out_hbm.at[cid].at[pl.ds(16 * _RPT, N - 16 * _RPT)])

    return k(h, e, src, dst)


def kernel(x, edge_index, edge_attr, batch, externals, W_node, b_node,
           We1, be1, We2, be2, Wc1, bc1, Wc2, bc2, gamma, beta,
           Wx1, bx1, Wx2, bx2, Wf1, bf1, Wf2, bf2):
    src = edge_index[0]
    dst = edge_index[1]
    h = _node_embed(x, W_node, b_node)
    e = _edge_mlp(edge_attr, We1, be1, We2, be2)
    for l in range(Wc1.shape[0]):
        parts = _edge_stage(h, e, src, dst)
        t, stats = _node_layer(h, parts[0], parts[1],
                               Wc1[l], bc1[l], Wc2[l], bc2[l])
        h = _bn_relu(t, stats, gamma[l], beta[l])
    out = _final(h, batch, externals,
                 Wx1, bx1, Wx2, bx2, Wf1, bf1, Wf2, bf2)
    return out[:, 0]
